# baseline (speedup 1.0000x reference)
import functools

import jax
import jax.numpy as jnp
from jax import lax
from jax.experimental import pallas as pl
from jax.experimental.pallas import tpu as pltpu, tpu_sc as plsc

H = 1024
E = 8
K = 2
V = 32000
F = 4096
S = 2048

_NEG_INF = float("-inf")


def _sc_gather(table, idx, n_rows, d):
    info = plsc.get_sparse_core_info()
    nw = info.num_cores * info.num_subcores
    per_w = n_rows // nw
    mesh = plsc.VectorSubcoreMesh(core_axis_name="c", subcore_axis_name="s")

    @functools.partial(
        pl.kernel,
        mesh=mesh,
        out_type=jax.ShapeDtypeStruct((n_rows, d), jnp.float32),
        scratch_types=[
            pltpu.VMEM((per_w,), jnp.int32),
            pltpu.VMEM((per_w, d), jnp.float32),
            pltpu.SemaphoreType.DMA,
        ],
    )
    def k(table_hbm, idx_hbm, out_hbm, idx_v, rows_v, sem):
        wid = lax.axis_index("s") * info.num_cores + lax.axis_index("c")
        base = wid * per_w
        pltpu.sync_copy(idx_hbm.at[pl.ds(base, per_w)], idx_v)
        pltpu.async_copy(table_hbm.at[idx_v], rows_v, sem).wait()
        pltpu.sync_copy(rows_v, out_hbm.at[pl.ds(base, per_w)])

    return k(table, idx)


def _router_body(h_ref, wr_ref, br_ref, rw_ref, sel_ref):
    logits = jnp.dot(h_ref[...], wr_ref[...], preferred_element_type=jnp.float32)
    logits = logits + br_ref[...][None, :]
    col = lax.broadcasted_iota(jnp.int32, logits.shape, 1)
    valid = col < E
    logits = jnp.where(valid, logits, _NEG_INF)
    m1 = jnp.max(logits, axis=1, keepdims=True)
    a1 = jnp.min(jnp.where(logits == m1, col, logits.shape[1]), axis=1, keepdims=True)
    l2 = jnp.where(col == a1, _NEG_INF, logits)
    m2 = jnp.max(l2, axis=1, keepdims=True)
    a2 = jnp.min(jnp.where(l2 == m2, col, logits.shape[1]), axis=1, keepdims=True)
    e2 = jnp.exp(m2 - m1)
    denom = 1.0 + e2
    w1 = 1.0 / denom
    w2 = e2 / denom
    lane = lax.broadcasted_iota(jnp.int32, rw_ref.shape, 1)
    rw_ref[...] = jnp.where(lane == 0, w1, jnp.where(lane == 1, w2, 0.0))
    sel_ref[...] = jnp.where(lane == 0, a1, jnp.where(lane == 1, a2, 0))


def _router(hidden, wr_pad, br_pad):
    rw, sel = pl.pallas_call(
        _router_body,
        out_shape=(
            jax.ShapeDtypeStruct((S, 128), jnp.float32),
            jax.ShapeDtypeStruct((S, 128), jnp.int32),
        ),
    )(hidden, wr_pad, br_pad)
    return rw[:, :K], sel[:, :K]


_FC = 1024
_NFC = F // _FC


def _ffn_body(sel_ref, x_ref, w1_ref, b1_ref, w2_ref, b2_ref, out_ref, h1_ref):
    e = pl.program_id(0)
    fc = pl.program_id(1)

    @pl.when(jnp.logical_and(e == 0, fc == 0))
    def _():
        out_ref[...] = jnp.zeros_like(out_ref)

    x = x_ref[...].astype(jnp.bfloat16)
    h1 = jnp.dot(x, w1_ref[0].astype(jnp.bfloat16),
                 preferred_element_type=jnp.float32)
    h1 = h1 + b1_ref[0]
    h1 = h1 * 0.5 * (1.0 + lax.erf(h1 * (2.0 ** -0.5)))
    h2 = jnp.dot(h1.astype(jnp.bfloat16), w2_ref[0].astype(jnp.bfloat16),
                 preferred_element_type=jnp.float32)
    del h1_ref
    mask = jnp.any(sel_ref[...] == e, axis=1, keepdims=True).astype(jnp.float32)
    bias = jnp.where(fc == 0, 1.0, 0.0)
    h2 = h2 + bias * b2_ref[0]
    out_ref[...] += mask * h2


def _ffn_dense(hidden, sel, w1, b1, w2, b2):
    return pl.pallas_call(
        _ffn_body,
        grid=(E, _NFC),
        in_specs=[
            pl.BlockSpec((S, K), lambda e, fc: (0, 0)),
            pl.BlockSpec((S, H), lambda e, fc: (0, 0)),
            pl.BlockSpec((1, H, _FC), lambda e, fc: (e, 0, fc)),
            pl.BlockSpec((1, 1, _FC), lambda e, fc: (e, 0, fc)),
            pl.BlockSpec((1, _FC, H), lambda e, fc: (e, fc, 0)),
            pl.BlockSpec((1, 1, H), lambda e, fc: (e, 0, 0)),
        ],
        out_specs=pl.BlockSpec((S, H), lambda e, fc: (0, 0)),
        out_shape=jax.ShapeDtypeStruct((S, H), jnp.float32),
        scratch_shapes=[pltpu.VMEM((S, _FC), jnp.float32)],
    )(sel, hidden, w1, b1.reshape(E, 1, F), w2, b2.reshape(E, 1, H))


_VC = 1280
_NVC = V // _VC


def _lm_body(h_ref, w_ref, b_ref, out_ref):
    out_ref[...] = (
        jnp.dot(h_ref[...].astype(jnp.bfloat16), w_ref[...].astype(jnp.bfloat16),
                preferred_element_type=jnp.float32)
        + b_ref[...]
    )


def _lm_head(hidden_out, wlm, blm):
    return pl.pallas_call(
        _lm_body,
        grid=(_NVC,),
        in_specs=[
            pl.BlockSpec((S, H), lambda v: (0, 0)),
            pl.BlockSpec((H, _VC), lambda v: (0, v)),
            pl.BlockSpec((1, _VC), lambda v: (0, v)),
        ],
        out_specs=pl.BlockSpec((S, _VC), lambda v: (0, v)),
        out_shape=jax.ShapeDtypeStruct((S, V), jnp.float32),
    )(hidden_out, wlm, blm.reshape(1, V))


def kernel(input_ids, emb, Wr, br, W1, b1, W2, b2, Wlm, blm):
    batch, seq = input_ids.shape
    ids = input_ids.reshape(-1).astype(jnp.int32)

    hidden = _sc_gather(emb, ids, S, H)

    wr_pad = jnp.zeros((H, 128), jnp.float32).at[:, :E].set(Wr)
    br_pad = jnp.zeros((128,), jnp.float32).at[:E].set(br)
    rw, sel = _router(hidden, wr_pad, br_pad)

    hidden_out = _ffn_dense(hidden, sel, W1, b1, W2, b2)

    logits = _lm_head(hidden_out, Wlm, blm)
    return (logits.reshape(batch, seq, V), rw, sel)


# speedup vs baseline: 1.1539x; 1.1539x over previous
import functools

import jax
import jax.numpy as jnp
from jax import lax
from jax.experimental import pallas as pl
from jax.experimental.pallas import tpu as pltpu, tpu_sc as plsc

H = 1024
E = 8
K = 2
V = 32000
F = 4096
S = 2048

_NEG_INF = float("-inf")


def _sc_gather(table, idx, n_rows, d):
    info = plsc.get_sparse_core_info()
    nw = info.num_cores * info.num_subcores
    per_w = n_rows // nw
    mesh = plsc.VectorSubcoreMesh(core_axis_name="c", subcore_axis_name="s")

    @functools.partial(
        pl.kernel,
        mesh=mesh,
        out_type=jax.ShapeDtypeStruct((n_rows, d), jnp.float32),
        scratch_types=[
            pltpu.VMEM((per_w,), jnp.int32),
            pltpu.VMEM((per_w, d), jnp.float32),
            pltpu.SemaphoreType.DMA,
        ],
    )
    def k(table_hbm, idx_hbm, out_hbm, idx_v, rows_v, sem):
        wid = lax.axis_index("s") * info.num_cores + lax.axis_index("c")
        base = wid * per_w
        pltpu.sync_copy(idx_hbm.at[pl.ds(base, per_w)], idx_v)
        pltpu.async_copy(table_hbm.at[idx_v], rows_v, sem).wait()
        pltpu.sync_copy(rows_v, out_hbm.at[pl.ds(base, per_w)])

    return k(table, idx)


def _router_body(h_ref, wr_ref, br_ref, rw_ref, sel_ref):
    logits = jnp.dot(h_ref[...], wr_ref[...], preferred_element_type=jnp.float32)
    logits = logits + br_ref[...][None, :]
    col = lax.broadcasted_iota(jnp.int32, logits.shape, 1)
    valid = col < E
    logits = jnp.where(valid, logits, _NEG_INF)
    m1 = jnp.max(logits, axis=1, keepdims=True)
    a1 = jnp.min(jnp.where(logits == m1, col, logits.shape[1]), axis=1, keepdims=True)
    l2 = jnp.where(col == a1, _NEG_INF, logits)
    m2 = jnp.max(l2, axis=1, keepdims=True)
    a2 = jnp.min(jnp.where(l2 == m2, col, logits.shape[1]), axis=1, keepdims=True)
    e2 = jnp.exp(m2 - m1)
    denom = 1.0 + e2
    w1 = 1.0 / denom
    w2 = e2 / denom
    lane = lax.broadcasted_iota(jnp.int32, rw_ref.shape, 1)
    rw_ref[...] = jnp.where(lane == 0, w1, jnp.where(lane == 1, w2, 0.0))
    sel_ref[...] = jnp.where(lane == 0, a1, jnp.where(lane == 1, a2, 0))


def _router(hidden, wr_pad, br_pad):
    rw, sel = pl.pallas_call(
        _router_body,
        out_shape=(
            jax.ShapeDtypeStruct((S, 128), jnp.float32),
            jax.ShapeDtypeStruct((S, 128), jnp.int32),
        ),
    )(hidden, wr_pad, br_pad)
    return rw[:, :K], sel[:, :K]


_FC = 1024
_NFC = F // _FC


def _ffn_body(sel_ref, x_ref, w1_ref, b1_ref, w2_ref, b2_ref, out_ref, h1_ref):
    e = pl.program_id(0)
    fc = pl.program_id(1)

    @pl.when(jnp.logical_and(e == 0, fc == 0))
    def _():
        out_ref[...] = jnp.zeros_like(out_ref)

    x = x_ref[...].astype(jnp.bfloat16)
    h1 = jnp.dot(x, w1_ref[0].astype(jnp.bfloat16),
                 preferred_element_type=jnp.float32)
    h1 = h1 + b1_ref[0]
    h1 = h1 * 0.5 * (1.0 + lax.erf(h1 * (2.0 ** -0.5)))
    h2 = jnp.dot(h1.astype(jnp.bfloat16), w2_ref[0].astype(jnp.bfloat16),
                 preferred_element_type=jnp.float32)
    del h1_ref
    mask = jnp.any(sel_ref[...] == e, axis=1, keepdims=True).astype(jnp.float32)
    bias = jnp.where(fc == 0, 1.0, 0.0)
    h2 = h2 + bias * b2_ref[0]
    out_ref[...] += mask * h2


def _ffn_dense(hidden, sel, w1, b1, w2, b2):
    return pl.pallas_call(
        _ffn_body,
        grid=(E, _NFC),
        in_specs=[
            pl.BlockSpec((S, K), lambda e, fc: (0, 0)),
            pl.BlockSpec((S, H), lambda e, fc: (0, 0)),
            pl.BlockSpec((1, H, _FC), lambda e, fc: (e, 0, fc)),
            pl.BlockSpec((1, 1, _FC), lambda e, fc: (e, 0, fc)),
            pl.BlockSpec((1, _FC, H), lambda e, fc: (e, fc, 0)),
            pl.BlockSpec((1, 1, H), lambda e, fc: (e, 0, 0)),
        ],
        out_specs=pl.BlockSpec((S, H), lambda e, fc: (0, 0)),
        out_shape=jax.ShapeDtypeStruct((S, H), jnp.float32),
        scratch_shapes=[pltpu.VMEM((S, _FC), jnp.float32)],
    )(sel, hidden, w1, b1.reshape(E, 1, F), w2, b2.reshape(E, 1, H))


_VC = 1280
_NVC = V // _VC


def _lm_body(h_ref, w_ref, b_ref, out_ref):
    out_ref[...] = (
        jnp.dot(h_ref[...].astype(jnp.bfloat16), w_ref[...].astype(jnp.bfloat16),
                preferred_element_type=jnp.float32)
        + b_ref[...]
    )


def _lm_head(hidden_out, wlm, blm):
    return pl.pallas_call(
        _lm_body,
        grid=(_NVC,),
        in_specs=[
            pl.BlockSpec((S, H), lambda v: (0, 0)),
            pl.BlockSpec((H, _VC), lambda v: (0, v)),
            pl.BlockSpec((1, _VC), lambda v: (0, v)),
        ],
        out_specs=pl.BlockSpec((S, _VC), lambda v: (0, v)),
        out_shape=jax.ShapeDtypeStruct((S, V), jnp.float32),
    )(hidden_out, wlm, blm.reshape(1, V))


_T = 128
_NP = S * K
_NT = _NP // _T + E
_NROWS = _NT * _T


def _dispatch_plan(sel):
    ef = sel.reshape(-1).astype(jnp.int32)
    oh = (ef[:, None] == jnp.arange(E, dtype=jnp.int32)[None, :]).astype(jnp.int32)
    cnt = oh.sum(axis=0)
    rank = (jnp.cumsum(oh, axis=0) - oh)
    rank_j = (rank * oh).sum(axis=1)
    padded = ((cnt + _T - 1) // _T) * _T
    ends = jnp.cumsum(padded)
    pstart = ends - padded
    d = jnp.take(pstart, ef) + rank_j
    d2 = d.reshape(S, K)
    d0 = d2[:, 0]
    d1 = d2[:, 1]
    tile_starts = jnp.arange(_NT, dtype=jnp.int32) * _T
    tile_eid = (tile_starts[:, None] >= ends[None, :]).sum(axis=1)
    tile_eid = jnp.minimum(tile_eid, E - 1).astype(jnp.int32)
    n_tiles = (ends[-1] // _T).astype(jnp.int32)
    meta = jnp.concatenate([tile_eid, n_tiles[None]])
    return d0, d1, meta


def _sc_dispatch(hidden, d0, d1):
    info = plsc.get_sparse_core_info()
    nw = info.num_cores * info.num_subcores
    per_w = S // nw
    mesh = plsc.VectorSubcoreMesh(core_axis_name="c", subcore_axis_name="s")

    @functools.partial(
        pl.kernel,
        mesh=mesh,
        out_type=jax.ShapeDtypeStruct((_NROWS, H), jnp.float32),
        scratch_types=[
            pltpu.VMEM((per_w,), jnp.int32),
            pltpu.VMEM((per_w,), jnp.int32),
            pltpu.VMEM((per_w, H), jnp.float32),
            pltpu.SemaphoreType.DMA,
        ],
    )
    def k(hid_hbm, d0_hbm, d1_hbm, out_hbm, i0_v, i1_v, rows_v, sem):
        wid = lax.axis_index("s") * info.num_cores + lax.axis_index("c")
        base = wid * per_w
        pltpu.sync_copy(hid_hbm.at[pl.ds(base, per_w)], rows_v)
        pltpu.sync_copy(d0_hbm.at[pl.ds(base, per_w)], i0_v)
        pltpu.sync_copy(d1_hbm.at[pl.ds(base, per_w)], i1_v)
        pltpu.async_copy(rows_v, out_hbm.at[i0_v], sem).wait()
        pltpu.async_copy(rows_v, out_hbm.at[i1_v], sem).wait()

    return k(hidden, d0, d1)


def _sc_combine(ffn_out, d0, d1):
    info = plsc.get_sparse_core_info()
    nw = info.num_cores * info.num_subcores
    per_w = S // nw
    half = per_w // 2
    mesh = plsc.VectorSubcoreMesh(core_axis_name="c", subcore_axis_name="s")

    @functools.partial(
        pl.kernel,
        mesh=mesh,
        out_type=jax.ShapeDtypeStruct((S, H), jnp.float32),
        scratch_types=[
            pltpu.VMEM((per_w,), jnp.int32),
            pltpu.VMEM((half,), jnp.int32),
            pltpu.VMEM((per_w, H), jnp.float32),
            pltpu.VMEM((half, H), jnp.float32),
            pltpu.SemaphoreType.DMA,
        ],
    )
    def k(src_hbm, d0_hbm, d1_hbm, out_hbm, i0_v, i1_v, a_v, b_v, sem):
        wid = lax.axis_index("s") * info.num_cores + lax.axis_index("c")
        base = wid * per_w
        pltpu.sync_copy(d0_hbm.at[pl.ds(base, per_w)], i0_v)
        pltpu.async_copy(src_hbm.at[i0_v], a_v, sem).wait()
        for c in range(2):
            pltpu.sync_copy(d1_hbm.at[pl.ds(base + c * half, half)], i1_v)
            pltpu.async_copy(src_hbm.at[i1_v], b_v, sem).wait()

            def row_body(r, _):
                def col_body(cc, __):
                    for u in range(4):
                        off = cc * 64 + u * 16
                        a_v[c * half + r, pl.ds(off, 16)] += b_v[r, pl.ds(off, 16)]
                    return __
                return lax.fori_loop(0, H // 64, col_body, _)

            lax.fori_loop(0, half, row_body, 0)
        pltpu.sync_copy(a_v, out_hbm.at[pl.ds(base, per_w)])

    return k(ffn_out, d0, d1)


def _k1_body(meta_ref, x_ref, w1_ref, b1_ref, h1_ref):
    t = pl.program_id(0)

    @pl.when(t < meta_ref[_NT])
    def _():
        x = x_ref[...].astype(jnp.bfloat16)
        h1 = jnp.dot(x, w1_ref[0].astype(jnp.bfloat16),
                     preferred_element_type=jnp.float32)
        h1 = h1 + b1_ref[0]
        h1 = h1 * 0.5 * (1.0 + lax.erf(h1 * (2.0 ** -0.5)))
        h1_ref[...] = h1.astype(jnp.bfloat16)


def _k2_body(meta_ref, h1_ref, w2_ref, b2_ref, out_ref):
    t = pl.program_id(0)

    @pl.when(t < meta_ref[_NT])
    def _():
        h2 = jnp.dot(h1_ref[...], w2_ref[0].astype(jnp.bfloat16),
                     preferred_element_type=jnp.float32)
        out_ref[...] = h2 + b2_ref[0]


def _ffn_grouped(xdisp, meta, w1, b1, w2, b2):
    h1 = pl.pallas_call(
        _k1_body,
        grid_spec=pltpu.PrefetchScalarGridSpec(
            num_scalar_prefetch=1,
            grid=(_NT,),
            in_specs=[
                pl.BlockSpec((_T, H), lambda t, m: (t, 0)),
                pl.BlockSpec((1, H, F), lambda t, m: (m[t], 0, 0)),
                pl.BlockSpec((1, 1, F), lambda t, m: (m[t], 0, 0)),
            ],
            out_specs=pl.BlockSpec((_T, F), lambda t, m: (t, 0)),
        ),
        out_shape=jax.ShapeDtypeStruct((_NROWS, F), jnp.bfloat16),
    )(meta, xdisp, w1, b1.reshape(E, 1, F))
    return pl.pallas_call(
        _k2_body,
        grid_spec=pltpu.PrefetchScalarGridSpec(
            num_scalar_prefetch=1,
            grid=(_NT,),
            in_specs=[
                pl.BlockSpec((_T, F), lambda t, m: (t, 0)),
                pl.BlockSpec((1, F, H), lambda t, m: (m[t], 0, 0)),
                pl.BlockSpec((1, 1, H), lambda t, m: (m[t], 0, 0)),
            ],
            out_specs=pl.BlockSpec((_T, H), lambda t, m: (t, 0)),
        ),
        out_shape=jax.ShapeDtypeStruct((_NROWS, H), jnp.float32),
    )(meta, h1, w2, b2.reshape(E, 1, H))


def kernel(input_ids, emb, Wr, br, W1, b1, W2, b2, Wlm, blm):
    batch, seq = input_ids.shape
    ids = input_ids.reshape(-1).astype(jnp.int32)

    hidden = _sc_gather(emb, ids, S, H)

    wr_pad = jnp.zeros((H, 128), jnp.float32).at[:, :E].set(Wr)
    br_pad = jnp.zeros((128,), jnp.float32).at[:E].set(br)
    rw, sel = _router(hidden, wr_pad, br_pad)

    d0, d1, meta = _dispatch_plan(sel)
    xdisp = _sc_dispatch(hidden, d0, d1)
    ffn_out = _ffn_grouped(xdisp, meta, W1, b1, W2, b2)
    hidden_out = _sc_combine(ffn_out, d0, d1)

    logits = _lm_head(hidden_out, Wlm, blm)
    return (logits.reshape(batch, seq, V), rw, sel)


# speedup vs baseline: 1.1854x; 1.0273x over previous
import functools

import jax
import jax.numpy as jnp
from jax import lax
from jax.experimental import pallas as pl
from jax.experimental.pallas import tpu as pltpu, tpu_sc as plsc

H = 1024
E = 8
K = 2
V = 32000
F = 4096
S = 2048

_NEG_INF = float("-inf")


def _sc_gather(table, idx, n_rows, d):
    info = plsc.get_sparse_core_info()
    nw = info.num_cores * info.num_subcores
    per_w = n_rows // nw
    mesh = plsc.VectorSubcoreMesh(core_axis_name="c", subcore_axis_name="s")

    @functools.partial(
        pl.kernel,
        mesh=mesh,
        out_type=jax.ShapeDtypeStruct((n_rows, d), jnp.float32),
        scratch_types=[
            pltpu.VMEM((per_w,), jnp.int32),
            pltpu.VMEM((per_w, d), jnp.float32),
            pltpu.SemaphoreType.DMA,
        ],
    )
    def k(table_hbm, idx_hbm, out_hbm, idx_v, rows_v, sem):
        wid = lax.axis_index("s") * info.num_cores + lax.axis_index("c")
        base = wid * per_w
        pltpu.sync_copy(idx_hbm.at[pl.ds(base, per_w)], idx_v)
        pltpu.async_copy(table_hbm.at[idx_v], rows_v, sem).wait()
        pltpu.sync_copy(rows_v, out_hbm.at[pl.ds(base, per_w)])

    return k(table, idx)


def _router_body(h_ref, wr_ref, br_ref, rw_ref, sel_ref, d0_ref, d1_ref, meta_ref):
    logits = jnp.dot(h_ref[...], wr_ref[...], preferred_element_type=jnp.float32)
    logits = logits + br_ref[...][None, :]
    col = lax.broadcasted_iota(jnp.int32, logits.shape, 1)
    valid = col < E
    logits = jnp.where(valid, logits, _NEG_INF)
    m1 = jnp.max(logits, axis=1, keepdims=True)
    a1 = jnp.min(jnp.where(logits == m1, col, logits.shape[1]), axis=1, keepdims=True)
    l2 = jnp.where(col == a1, _NEG_INF, logits)
    m2 = jnp.max(l2, axis=1, keepdims=True)
    a2 = jnp.min(jnp.where(l2 == m2, col, logits.shape[1]), axis=1, keepdims=True)
    e2 = jnp.exp(m2 - m1)
    denom = 1.0 + e2
    w1 = 1.0 / denom
    w2 = e2 / denom
    lane = lax.broadcasted_iota(jnp.int32, rw_ref.shape, 1)
    rw_ref[...] = jnp.where(lane == 0, w1, jnp.where(lane == 1, w2, 0.0))
    sel_ref[...] = jnp.where(lane == 0, a1, jnp.where(lane == 1, a2, 0))

    ind = jnp.logical_or(col == a1, col == a2).astype(jnp.int32)
    cum = ind
    sh = 1
    while sh < S:
        shifted = jnp.concatenate(
            [jnp.zeros((sh, cum.shape[1]), jnp.int32), cum[:-sh, :]], axis=0)
        cum = cum + shifted
        sh *= 2
    excl = cum - ind
    counts = cum[S - 1:S, :]
    padded = ((counts + (_T - 1)) // _T) * _T
    col1 = lax.broadcasted_iota(jnp.int32, (1, 128), 1)
    ends = jnp.zeros((1, 128), jnp.int32)
    run = jnp.zeros((1, 1), jnp.int32)
    for e in range(E):
        pe = jnp.sum(jnp.where(col1 == e, padded, 0), axis=1, keepdims=True)
        run = run + pe
        ends = jnp.where(col1 == e, run, ends)
    pstart = ends - padded
    base0 = jnp.sum(jnp.where(col == a1, pstart, 0), axis=1, keepdims=True)
    r0 = jnp.sum(jnp.where(col == a1, excl, 0), axis=1, keepdims=True)
    base1 = jnp.sum(jnp.where(col == a2, pstart, 0), axis=1, keepdims=True)
    r1 = jnp.sum(jnp.where(col == a2, excl, 0), axis=1, keepdims=True)
    d0_ref[...] = base0 + r0
    d1_ref[...] = base1 + r1

    colm = lax.broadcasted_iota(jnp.int32, (1, 64), 1)
    tstart = colm * _T
    eid = jnp.zeros((1, 64), jnp.int32)
    for e in range(E):
        ends_e = jnp.sum(jnp.where(col1 == e, ends, 0), axis=1, keepdims=True)
        eid = eid + (tstart >= ends_e).astype(jnp.int32)
    eid = jnp.minimum(eid, E - 1)
    ordi = jnp.zeros((1, 64), jnp.int32)
    nxt = jnp.full((1, 64), -1, jnp.int32)
    for e in range(E):
        cnt_e = jnp.sum(jnp.where(col1 == e, counts, 0), axis=1, keepdims=True)
        present = (cnt_e > 0).astype(jnp.int32)
        ordi = ordi + present * (eid > e).astype(jnp.int32)
        ee = E - 1 - e
        cnt_ee = jnp.sum(jnp.where(col1 == ee, counts, 0), axis=1, keepdims=True)
        nxt = jnp.where(jnp.logical_and(cnt_ee > 0, ee > eid), ee, nxt)
    n_tiles = run // _T
    rowm = lax.broadcasted_iota(jnp.int32, (4, 64), 0)
    colm4 = lax.broadcasted_iota(jnp.int32, (4, 64), 1)
    meta = jnp.where(rowm == 0, eid,
                     jnp.where(rowm == 1, ordi,
                               jnp.where(rowm == 2, nxt, n_tiles)))
    del colm4
    meta_ref[...] = meta


def _router(hidden, wr_pad, br_pad):
    rw, sel, d0, d1, meta = pl.pallas_call(
        _router_body,
        out_shape=(
            jax.ShapeDtypeStruct((S, 128), jnp.float32),
            jax.ShapeDtypeStruct((S, 128), jnp.int32),
            jax.ShapeDtypeStruct((S, 1), jnp.int32),
            jax.ShapeDtypeStruct((S, 1), jnp.int32),
            jax.ShapeDtypeStruct((4, 64), jnp.int32),
        ),
    )(hidden, wr_pad, br_pad)
    return rw[:, :K], sel[:, :K], d0.reshape(S), d1.reshape(S), meta


_FC = 1024
_NFC = F // _FC


def _ffn_body(sel_ref, x_ref, w1_ref, b1_ref, w2_ref, b2_ref, out_ref, h1_ref):
    e = pl.program_id(0)
    fc = pl.program_id(1)

    @pl.when(jnp.logical_and(e == 0, fc == 0))
    def _():
        out_ref[...] = jnp.zeros_like(out_ref)

    x = x_ref[...].astype(jnp.bfloat16)
    h1 = jnp.dot(x, w1_ref[0].astype(jnp.bfloat16),
                 preferred_element_type=jnp.float32)
    h1 = h1 + b1_ref[0]
    h1 = h1 * 0.5 * (1.0 + lax.erf(h1 * (2.0 ** -0.5)))
    h2 = jnp.dot(h1.astype(jnp.bfloat16), w2_ref[0].astype(jnp.bfloat16),
                 preferred_element_type=jnp.float32)
    del h1_ref
    mask = jnp.any(sel_ref[...] == e, axis=1, keepdims=True).astype(jnp.float32)
    bias = jnp.where(fc == 0, 1.0, 0.0)
    h2 = h2 + bias * b2_ref[0]
    out_ref[...] += mask * h2


def _ffn_dense(hidden, sel, w1, b1, w2, b2):
    return pl.pallas_call(
        _ffn_body,
        grid=(E, _NFC),
        in_specs=[
            pl.BlockSpec((S, K), lambda e, fc: (0, 0)),
            pl.BlockSpec((S, H), lambda e, fc: (0, 0)),
            pl.BlockSpec((1, H, _FC), lambda e, fc: (e, 0, fc)),
            pl.BlockSpec((1, 1, _FC), lambda e, fc: (e, 0, fc)),
            pl.BlockSpec((1, _FC, H), lambda e, fc: (e, fc, 0)),
            pl.BlockSpec((1, 1, H), lambda e, fc: (e, 0, 0)),
        ],
        out_specs=pl.BlockSpec((S, H), lambda e, fc: (0, 0)),
        out_shape=jax.ShapeDtypeStruct((S, H), jnp.float32),
        scratch_shapes=[pltpu.VMEM((S, _FC), jnp.float32)],
    )(sel, hidden, w1, b1.reshape(E, 1, F), w2, b2.reshape(E, 1, H))


_VC = 1280
_NVC = V // _VC


def _lm_body(h_ref, w_ref, b_ref, out_ref):
    out_ref[...] = (
        jnp.dot(h_ref[...].astype(jnp.bfloat16), w_ref[...].astype(jnp.bfloat16),
                preferred_element_type=jnp.float32)
        + b_ref[...]
    )


def _lm_head(hidden_out, wlm, blm):
    return pl.pallas_call(
        _lm_body,
        grid=(_NVC,),
        in_specs=[
            pl.BlockSpec((S, H), lambda v: (0, 0)),
            pl.BlockSpec((H, _VC), lambda v: (0, v)),
            pl.BlockSpec((1, _VC), lambda v: (0, v)),
        ],
        out_specs=pl.BlockSpec((S, _VC), lambda v: (0, v)),
        out_shape=jax.ShapeDtypeStruct((S, V), jnp.float32),
    )(hidden_out, wlm, blm.reshape(1, V))


_T = 128
_NP = S * K
_NT = _NP // _T + E
_NROWS = _NT * _T


def _sc_dispatch(hidden, d0, d1):
    info = plsc.get_sparse_core_info()
    nw = info.num_cores * info.num_subcores
    per_w = S // nw
    mesh = plsc.VectorSubcoreMesh(core_axis_name="c", subcore_axis_name="s")

    @functools.partial(
        pl.kernel,
        mesh=mesh,
        out_type=jax.ShapeDtypeStruct((_NROWS, H), jnp.float32),
        scratch_types=[
            pltpu.VMEM((per_w,), jnp.int32),
            pltpu.VMEM((per_w,), jnp.int32),
            pltpu.VMEM((per_w, H), jnp.float32),
            pltpu.SemaphoreType.DMA,
        ],
    )
    def k(hid_hbm, d0_hbm, d1_hbm, out_hbm, i0_v, i1_v, rows_v, sem):
        wid = lax.axis_index("s") * info.num_cores + lax.axis_index("c")
        base = wid * per_w
        pltpu.sync_copy(hid_hbm.at[pl.ds(base, per_w)], rows_v)
        pltpu.sync_copy(d0_hbm.at[pl.ds(base, per_w)], i0_v)
        pltpu.sync_copy(d1_hbm.at[pl.ds(base, per_w)], i1_v)
        pltpu.async_copy(rows_v, out_hbm.at[i0_v], sem).wait()
        pltpu.async_copy(rows_v, out_hbm.at[i1_v], sem).wait()

    return k(hidden, d0, d1)


def _sc_combine(ffn_out, d0, d1):
    info = plsc.get_sparse_core_info()
    nw = info.num_cores * info.num_subcores
    per_w = S // nw
    half = per_w // 2
    mesh = plsc.VectorSubcoreMesh(core_axis_name="c", subcore_axis_name="s")

    @functools.partial(
        pl.kernel,
        mesh=mesh,
        out_type=jax.ShapeDtypeStruct((S, H), jnp.float32),
        scratch_types=[
            pltpu.VMEM((per_w,), jnp.int32),
            pltpu.VMEM((half,), jnp.int32),
            pltpu.VMEM((per_w, H), jnp.float32),
            pltpu.VMEM((half, H), jnp.float32),
            pltpu.SemaphoreType.DMA,
        ],
    )
    def k(src_hbm, d0_hbm, d1_hbm, out_hbm, i0_v, i1_v, a_v, b_v, sem):
        wid = lax.axis_index("s") * info.num_cores + lax.axis_index("c")
        base = wid * per_w
        pltpu.sync_copy(d0_hbm.at[pl.ds(base, per_w)], i0_v)
        pltpu.async_copy(src_hbm.at[i0_v], a_v, sem).wait()
        for c in range(2):
            pltpu.sync_copy(d1_hbm.at[pl.ds(base + c * half, half)], i1_v)
            pltpu.async_copy(src_hbm.at[i1_v], b_v, sem).wait()

            def row_body(r, _):
                def col_body(cc, __):
                    for u in range(4):
                        off = cc * 64 + u * 16
                        a_v[c * half + r, pl.ds(off, 16)] += b_v[r, pl.ds(off, 16)]
                    return __
                return lax.fori_loop(0, H // 64, col_body, _)

            lax.fori_loop(0, half, row_body, 0)
        pltpu.sync_copy(a_v, out_hbm.at[pl.ds(base, per_w)])

    return k(ffn_out, d0, d1)


def _k1_body(meta_ref, x_ref, w1_ref, b1_ref, h1_ref):
    t = pl.program_id(0)

    @pl.when(t < meta_ref[3, 0])
    def _():
        x = x_ref[...].astype(jnp.bfloat16)
        h1 = jnp.dot(x, w1_ref[0].astype(jnp.bfloat16),
                     preferred_element_type=jnp.float32)
        h1 = h1 + b1_ref[0]
        h1 = h1 * 0.5 * (1.0 + lax.erf(h1 * (2.0 ** -0.5)))
        h1_ref[...] = h1.astype(jnp.bfloat16)


def _k2_body(meta_ref, h1_ref, w2_ref, b2_ref, out_ref):
    t = pl.program_id(0)

    @pl.when(t < meta_ref[3, 0])
    def _():
        h2 = jnp.dot(h1_ref[...], w2_ref[0].astype(jnp.bfloat16),
                     preferred_element_type=jnp.float32)
        out_ref[...] = h2 + b2_ref[0]


def _ffn_grouped(xdisp, meta, w1, b1, w2, b2):
    h1 = pl.pallas_call(
        _k1_body,
        grid_spec=pltpu.PrefetchScalarGridSpec(
            num_scalar_prefetch=1,
            grid=(_NT,),
            in_specs=[
                pl.BlockSpec((_T, H), lambda t, m: (t, 0)),
                pl.BlockSpec((1, H, F), lambda t, m: (m[0, t], 0, 0)),
                pl.BlockSpec((1, 1, F), lambda t, m: (m[0, t], 0, 0)),
            ],
            out_specs=pl.BlockSpec((_T, F), lambda t, m: (t, 0)),
        ),
        out_shape=jax.ShapeDtypeStruct((_NROWS, F), jnp.bfloat16),
    )(meta, xdisp, w1, b1.reshape(E, 1, F))
    return pl.pallas_call(
        _k2_body,
        grid_spec=pltpu.PrefetchScalarGridSpec(
            num_scalar_prefetch=1,
            grid=(_NT,),
            in_specs=[
                pl.BlockSpec((_T, F), lambda t, m: (t, 0)),
                pl.BlockSpec((1, F, H), lambda t, m: (m[0, t], 0, 0)),
                pl.BlockSpec((1, 1, H), lambda t, m: (m[0, t], 0, 0)),
            ],
            out_specs=pl.BlockSpec((_T, H), lambda t, m: (t, 0)),
        ),
        out_shape=jax.ShapeDtypeStruct((_NROWS, H), jnp.float32),
    )(meta, h1, w2, b2.reshape(E, 1, H))


def kernel(input_ids, emb, Wr, br, W1, b1, W2, b2, Wlm, blm):
    batch, seq = input_ids.shape
    ids = input_ids.reshape(-1).astype(jnp.int32)

    hidden = _sc_gather(emb, ids, S, H)

    wr_pad = jnp.zeros((H, 128), jnp.float32).at[:, :E].set(Wr)
    br_pad = jnp.zeros((128,), jnp.float32).at[:E].set(br)
    rw, sel, d0, d1, meta = _router(hidden, wr_pad, br_pad)

    xdisp = _sc_dispatch(hidden, d0, d1)
    ffn_out = _ffn_grouped(xdisp, meta, W1, b1, W2, b2)
    hidden_out = _sc_combine(ffn_out, d0, d1)

    logits = _lm_head(hidden_out, Wlm, blm)
    return (logits.reshape(batch, seq, V), rw, sel)


# speedup vs baseline: 1.2341x; 1.0410x over previous
import functools

import jax
import jax.numpy as jnp
from jax import lax
from jax.experimental import pallas as pl
from jax.experimental.pallas import tpu as pltpu, tpu_sc as plsc

H = 1024
E = 8
K = 2
V = 32000
F = 4096
S = 2048

_NEG_INF = float("-inf")


def _sc_gather(table, idx, n_rows, d):
    info = plsc.get_sparse_core_info()
    nw = info.num_cores * info.num_subcores
    per_w = n_rows // nw
    mesh = plsc.VectorSubcoreMesh(core_axis_name="c", subcore_axis_name="s")

    @functools.partial(
        pl.kernel,
        mesh=mesh,
        out_type=jax.ShapeDtypeStruct((n_rows, d), jnp.float32),
        scratch_types=[
            pltpu.VMEM((per_w,), jnp.int32),
            pltpu.VMEM((per_w, d), jnp.float32),
            pltpu.SemaphoreType.DMA,
        ],
    )
    def k(table_hbm, idx_hbm, out_hbm, idx_v, rows_v, sem):
        wid = lax.axis_index("s") * info.num_cores + lax.axis_index("c")
        base = wid * per_w
        pltpu.sync_copy(idx_hbm.at[pl.ds(base, per_w)], idx_v)
        pltpu.async_copy(table_hbm.at[idx_v], rows_v, sem).wait()
        pltpu.sync_copy(rows_v, out_hbm.at[pl.ds(base, per_w)])

    return k(table, idx)


def _router_body(h_ref, wr_ref, br_ref, rw_ref, sel_ref, d0_ref, d1_ref, meta_ref):
    logits = jnp.dot(h_ref[...], wr_ref[...], preferred_element_type=jnp.float32)
    logits = logits + br_ref[...][None, :]
    col = lax.broadcasted_iota(jnp.int32, logits.shape, 1)
    valid = col < E
    logits = jnp.where(valid, logits, _NEG_INF)
    m1 = jnp.max(logits, axis=1, keepdims=True)
    a1 = jnp.min(jnp.where(logits == m1, col, logits.shape[1]), axis=1, keepdims=True)
    l2 = jnp.where(col == a1, _NEG_INF, logits)
    m2 = jnp.max(l2, axis=1, keepdims=True)
    a2 = jnp.min(jnp.where(l2 == m2, col, logits.shape[1]), axis=1, keepdims=True)
    e2 = jnp.exp(m2 - m1)
    denom = 1.0 + e2
    w1 = 1.0 / denom
    w2 = e2 / denom
    lane = lax.broadcasted_iota(jnp.int32, rw_ref.shape, 1)
    rw_ref[...] = jnp.where(lane == 0, w1, jnp.where(lane == 1, w2, 0.0))
    sel_ref[...] = jnp.where(lane == 0, a1, jnp.where(lane == 1, a2, 0))

    ind = jnp.logical_or(col == a1, col == a2).astype(jnp.int32)
    cum = ind
    sh = 1
    while sh < S:
        shifted = jnp.concatenate(
            [jnp.zeros((sh, cum.shape[1]), jnp.int32), cum[:-sh, :]], axis=0)
        cum = cum + shifted
        sh *= 2
    excl = cum - ind
    counts = cum[S - 1:S, :]
    padded = ((counts + (_T - 1)) // _T) * _T
    col1 = lax.broadcasted_iota(jnp.int32, (1, 128), 1)
    ends = jnp.zeros((1, 128), jnp.int32)
    run = jnp.zeros((1, 1), jnp.int32)
    for e in range(E):
        pe = jnp.sum(jnp.where(col1 == e, padded, 0), axis=1, keepdims=True)
        run = run + pe
        ends = jnp.where(col1 == e, run, ends)
    pstart = ends - padded
    base0 = jnp.sum(jnp.where(col == a1, pstart, 0), axis=1, keepdims=True)
    r0 = jnp.sum(jnp.where(col == a1, excl, 0), axis=1, keepdims=True)
    base1 = jnp.sum(jnp.where(col == a2, pstart, 0), axis=1, keepdims=True)
    r1 = jnp.sum(jnp.where(col == a2, excl, 0), axis=1, keepdims=True)
    d0_ref[...] = base0 + r0
    d1_ref[...] = base1 + r1

    colm = lax.broadcasted_iota(jnp.int32, (1, 64), 1)
    tstart = colm * _T
    eid = jnp.zeros((1, 64), jnp.int32)
    for e in range(E):
        ends_e = jnp.sum(jnp.where(col1 == e, ends, 0), axis=1, keepdims=True)
        eid = eid + (tstart >= ends_e).astype(jnp.int32)
    eid = jnp.minimum(eid, E - 1)
    ordi = jnp.zeros((1, 64), jnp.int32)
    nxt = jnp.full((1, 64), -1, jnp.int32)
    for e in range(E):
        cnt_e = jnp.sum(jnp.where(col1 == e, counts, 0), axis=1, keepdims=True)
        present = (cnt_e > 0).astype(jnp.int32)
        ordi = ordi + present * (eid > e).astype(jnp.int32)
        ee = E - 1 - e
        cnt_ee = jnp.sum(jnp.where(col1 == ee, counts, 0), axis=1, keepdims=True)
        nxt = jnp.where(jnp.logical_and(cnt_ee > 0, ee > eid), ee, nxt)
    n_tiles = run // _T
    rowm = lax.broadcasted_iota(jnp.int32, (4, 64), 0)
    colm4 = lax.broadcasted_iota(jnp.int32, (4, 64), 1)
    meta = jnp.where(rowm == 0, eid,
                     jnp.where(rowm == 1, ordi,
                               jnp.where(rowm == 2, nxt, n_tiles)))
    del colm4
    meta_ref[...] = meta


def _router(hidden, wr_pad, br_pad):
    rw, sel, d0, d1, meta = pl.pallas_call(
        _router_body,
        out_shape=(
            jax.ShapeDtypeStruct((S, 128), jnp.float32),
            jax.ShapeDtypeStruct((S, 128), jnp.int32),
            jax.ShapeDtypeStruct((S, 1), jnp.int32),
            jax.ShapeDtypeStruct((S, 1), jnp.int32),
            jax.ShapeDtypeStruct((4, 64), jnp.int32),
        ),
    )(hidden, wr_pad, br_pad)
    return rw[:, :K], sel[:, :K], d0.reshape(S), d1.reshape(S), meta


_FC = 1024
_NFC = F // _FC


def _ffn_body(sel_ref, x_ref, w1_ref, b1_ref, w2_ref, b2_ref, out_ref, h1_ref):
    e = pl.program_id(0)
    fc = pl.program_id(1)

    @pl.when(jnp.logical_and(e == 0, fc == 0))
    def _():
        out_ref[...] = jnp.zeros_like(out_ref)

    x = x_ref[...].astype(jnp.bfloat16)
    h1 = jnp.dot(x, w1_ref[0].astype(jnp.bfloat16),
                 preferred_element_type=jnp.float32)
    h1 = h1 + b1_ref[0]
    h1 = h1 * 0.5 * (1.0 + lax.erf(h1 * (2.0 ** -0.5)))
    h2 = jnp.dot(h1.astype(jnp.bfloat16), w2_ref[0].astype(jnp.bfloat16),
                 preferred_element_type=jnp.float32)
    del h1_ref
    mask = jnp.any(sel_ref[...] == e, axis=1, keepdims=True).astype(jnp.float32)
    bias = jnp.where(fc == 0, 1.0, 0.0)
    h2 = h2 + bias * b2_ref[0]
    out_ref[...] += mask * h2


def _ffn_dense(hidden, sel, w1, b1, w2, b2):
    return pl.pallas_call(
        _ffn_body,
        grid=(E, _NFC),
        in_specs=[
            pl.BlockSpec((S, K), lambda e, fc: (0, 0)),
            pl.BlockSpec((S, H), lambda e, fc: (0, 0)),
            pl.BlockSpec((1, H, _FC), lambda e, fc: (e, 0, fc)),
            pl.BlockSpec((1, 1, _FC), lambda e, fc: (e, 0, fc)),
            pl.BlockSpec((1, _FC, H), lambda e, fc: (e, fc, 0)),
            pl.BlockSpec((1, 1, H), lambda e, fc: (e, 0, 0)),
        ],
        out_specs=pl.BlockSpec((S, H), lambda e, fc: (0, 0)),
        out_shape=jax.ShapeDtypeStruct((S, H), jnp.float32),
        scratch_shapes=[pltpu.VMEM((S, _FC), jnp.float32)],
    )(sel, hidden, w1, b1.reshape(E, 1, F), w2, b2.reshape(E, 1, H))


_VC = 1280
_NVC = V // _VC


def _lm_body(h_ref, w_ref, b_ref, out_ref):
    out_ref[...] = (
        jnp.dot(h_ref[...].astype(jnp.bfloat16), w_ref[...].astype(jnp.bfloat16),
                preferred_element_type=jnp.float32)
        + b_ref[...]
    )


def _lm_head(hidden_out, wlm, blm):
    return pl.pallas_call(
        _lm_body,
        grid=(_NVC,),
        in_specs=[
            pl.BlockSpec((S, H), lambda v: (0, 0)),
            pl.BlockSpec((H, _VC), lambda v: (0, v)),
            pl.BlockSpec((1, _VC), lambda v: (0, v)),
        ],
        out_specs=pl.BlockSpec((S, _VC), lambda v: (0, v)),
        out_shape=jax.ShapeDtypeStruct((S, V), jnp.float32),
    )(hidden_out, wlm, blm.reshape(1, V))


_T = 128
_NP = S * K
_NT = _NP // _T + E
_NROWS = _NT * _T


def _sc_dispatch(hidden, d0, d1):
    info = plsc.get_sparse_core_info()
    nw = info.num_cores * info.num_subcores
    per_w = S // nw
    mesh = plsc.VectorSubcoreMesh(core_axis_name="c", subcore_axis_name="s")

    @functools.partial(
        pl.kernel,
        mesh=mesh,
        out_type=jax.ShapeDtypeStruct((_NROWS, H), jnp.float32),
        scratch_types=[
            pltpu.VMEM((per_w,), jnp.int32),
            pltpu.VMEM((per_w,), jnp.int32),
            pltpu.VMEM((per_w, H), jnp.float32),
            pltpu.SemaphoreType.DMA,
        ],
    )
    def k(hid_hbm, d0_hbm, d1_hbm, out_hbm, i0_v, i1_v, rows_v, sem):
        wid = lax.axis_index("s") * info.num_cores + lax.axis_index("c")
        base = wid * per_w
        pltpu.sync_copy(hid_hbm.at[pl.ds(base, per_w)], rows_v)
        pltpu.sync_copy(d0_hbm.at[pl.ds(base, per_w)], i0_v)
        pltpu.sync_copy(d1_hbm.at[pl.ds(base, per_w)], i1_v)
        pltpu.async_copy(rows_v, out_hbm.at[i0_v], sem).wait()
        pltpu.async_copy(rows_v, out_hbm.at[i1_v], sem).wait()

    return k(hidden, d0, d1)


def _sc_combine(ffn_out, d0, d1):
    info = plsc.get_sparse_core_info()
    nw = info.num_cores * info.num_subcores
    per_w = S // nw
    half = per_w // 2
    mesh = plsc.VectorSubcoreMesh(core_axis_name="c", subcore_axis_name="s")

    @functools.partial(
        pl.kernel,
        mesh=mesh,
        out_type=jax.ShapeDtypeStruct((S, H), jnp.float32),
        scratch_types=[
            pltpu.VMEM((per_w,), jnp.int32),
            pltpu.VMEM((half,), jnp.int32),
            pltpu.VMEM((per_w, H), jnp.float32),
            pltpu.VMEM((half, H), jnp.float32),
            pltpu.SemaphoreType.DMA,
        ],
    )
    def k(src_hbm, d0_hbm, d1_hbm, out_hbm, i0_v, i1_v, a_v, b_v, sem):
        wid = lax.axis_index("s") * info.num_cores + lax.axis_index("c")
        base = wid * per_w
        pltpu.sync_copy(d0_hbm.at[pl.ds(base, per_w)], i0_v)
        pltpu.async_copy(src_hbm.at[i0_v], a_v, sem).wait()
        for c in range(2):
            pltpu.sync_copy(d1_hbm.at[pl.ds(base + c * half, half)], i1_v)
            pltpu.async_copy(src_hbm.at[i1_v], b_v, sem).wait()

            def row_body(r, _):
                def col_body(cc, __):
                    for u in range(4):
                        off = cc * 64 + u * 16
                        a_v[c * half + r, pl.ds(off, 16)] += b_v[r, pl.ds(off, 16)]
                    return __
                return lax.fori_loop(0, H // 64, col_body, _)

            lax.fori_loop(0, half, row_body, 0)
        pltpu.sync_copy(a_v, out_hbm.at[pl.ds(base, per_w)])

    return k(ffn_out, d0, d1)


def _weight_pipeline(meta_ref, w_hbm, wb0, wb1, sm0, sm1, compute):
    t = pl.program_id(0)
    eid = meta_ref[0, t]
    ordi = meta_ref[1, t]
    nxt = meta_ref[2, t]
    prev_e = meta_ref[0, jnp.maximum(t - 1, 0)]
    first = jnp.logical_or(t == 0, eid != prev_e)

    @pl.when(t == 0)
    def _():
        pltpu.make_async_copy(w_hbm.at[eid], wb0, sm0).start()

    @pl.when(jnp.logical_and(t == 0, nxt >= 0))
    def _():
        pltpu.make_async_copy(w_hbm.at[nxt], wb1, sm1).start()

    issue = jnp.logical_and(first, jnp.logical_and(t > 0, nxt >= 0))

    @pl.when(jnp.logical_and(issue, (ordi + 1) % 2 == 0))
    def _():
        pltpu.make_async_copy(w_hbm.at[nxt], wb0, sm0).start()

    @pl.when(jnp.logical_and(issue, (ordi + 1) % 2 == 1))
    def _():
        pltpu.make_async_copy(w_hbm.at[nxt], wb1, sm1).start()

    @pl.when(jnp.logical_and(first, ordi % 2 == 0))
    def _():
        pltpu.make_async_copy(w_hbm.at[eid], wb0, sm0).wait()

    @pl.when(jnp.logical_and(first, ordi % 2 == 1))
    def _():
        pltpu.make_async_copy(w_hbm.at[eid], wb1, sm1).wait()

    @pl.when(ordi % 2 == 0)
    def _():
        compute(wb0)

    @pl.when(ordi % 2 == 1)
    def _():
        compute(wb1)


def _k1_body(meta_ref, x_ref, w1_hbm, b1_ref, h1_ref, wb0, wb1, sm0, sm1):
    t = pl.program_id(0)

    @pl.when(t < meta_ref[3, 0])
    def _():
        def compute(wb):
            x = x_ref[...].astype(jnp.bfloat16)
            h1 = jnp.dot(x, wb[...].astype(jnp.bfloat16),
                         preferred_element_type=jnp.float32)
            h1 = h1 + b1_ref[0]
            h1 = h1 * 0.5 * (1.0 + lax.erf(h1 * (2.0 ** -0.5)))
            h1_ref[...] = h1.astype(jnp.bfloat16)

        _weight_pipeline(meta_ref, w1_hbm, wb0, wb1, sm0, sm1, compute)


def _k2_body(meta_ref, h1_ref, w2_hbm, b2_ref, out_ref, wb0, wb1, sm0, sm1):
    t = pl.program_id(0)

    @pl.when(t < meta_ref[3, 0])
    def _():
        def compute(wb):
            h2 = jnp.dot(h1_ref[...], wb[...].astype(jnp.bfloat16),
                         preferred_element_type=jnp.float32)
            out_ref[...] = h2 + b2_ref[0]

        _weight_pipeline(meta_ref, w2_hbm, wb0, wb1, sm0, sm1, compute)


def _ffn_grouped(xdisp, meta, w1, b1, w2, b2):
    h1 = pl.pallas_call(
        _k1_body,
        grid_spec=pltpu.PrefetchScalarGridSpec(
            num_scalar_prefetch=1,
            grid=(_NT,),
            in_specs=[
                pl.BlockSpec((_T, H), lambda t, m: (t, 0)),
                pl.BlockSpec(memory_space=pl.ANY),
                pl.BlockSpec((1, 1, F), lambda t, m: (m[0, t], 0, 0)),
            ],
            out_specs=pl.BlockSpec((_T, F), lambda t, m: (t, 0)),
            scratch_shapes=[
                pltpu.VMEM((H, F), jnp.float32),
                pltpu.VMEM((H, F), jnp.float32),
                pltpu.SemaphoreType.DMA,
                pltpu.SemaphoreType.DMA,
            ],
        ),
        out_shape=jax.ShapeDtypeStruct((_NROWS, F), jnp.bfloat16),
    )(meta, xdisp, w1, b1.reshape(E, 1, F))
    return pl.pallas_call(
        _k2_body,
        grid_spec=pltpu.PrefetchScalarGridSpec(
            num_scalar_prefetch=1,
            grid=(_NT,),
            in_specs=[
                pl.BlockSpec((_T, F), lambda t, m: (t, 0)),
                pl.BlockSpec(memory_space=pl.ANY),
                pl.BlockSpec((1, 1, H), lambda t, m: (m[0, t], 0, 0)),
            ],
            out_specs=pl.BlockSpec((_T, H), lambda t, m: (t, 0)),
            scratch_shapes=[
                pltpu.VMEM((F, H), jnp.float32),
                pltpu.VMEM((F, H), jnp.float32),
                pltpu.SemaphoreType.DMA,
                pltpu.SemaphoreType.DMA,
            ],
        ),
        out_shape=jax.ShapeDtypeStruct((_NROWS, H), jnp.float32),
    )(meta, h1, w2, b2.reshape(E, 1, H))


def kernel(input_ids, emb, Wr, br, W1, b1, W2, b2, Wlm, blm):
    batch, seq = input_ids.shape
    ids = input_ids.reshape(-1).astype(jnp.int32)

    hidden = _sc_gather(emb, ids, S, H)

    wr_pad = jnp.zeros((H, 128), jnp.float32).at[:, :E].set(Wr)
    br_pad = jnp.zeros((128,), jnp.float32).at[:E].set(br)
    rw, sel, d0, d1, meta = _router(hidden, wr_pad, br_pad)

    xdisp = _sc_dispatch(hidden, d0, d1)
    ffn_out = _ffn_grouped(xdisp, meta, W1, b1, W2, b2)
    hidden_out = _sc_combine(ffn_out, d0, d1)

    logits = _lm_head(hidden_out, Wlm, blm)
    return (logits.reshape(batch, seq, V), rw, sel)
